# baseline (speedup 1.0000x reference)
import functools

import jax
import jax.numpy as jnp
from jax import lax
from jax.experimental import pallas as pl
from jax.experimental.pallas import tpu as pltpu
from jax.experimental.pallas import tpu_sc as plsc

N = 10000
D = 128
H = 256
C = 64
E = 320000

NC = 2
NS = 16
NW = NC * NS

PB = 320
N2 = NW * PB
W = 2048
EPAD = ((E + W - 1) // W) * W
NWIN = EPAD // W
WC = 2560

_NEG = -3.0e38


def _unpack_window(mpk, msrc, mdl, cnt, nch, rb, iota, pad_rows):
    def unpack_body(u, _):
        gidx = 16 * u + iota
        p = mpk[pl.ds(16 * u, 16)]
        valid = gidx < cnt
        msrc[pl.ds(16 * u, 16)] = jnp.where(
            valid, lax.shift_right_logical(p, 9), pad_rows)
        mdl[pl.ds(16 * u, 16)] = jnp.where(valid, p & 511, PB)
        return 0

    lax.fori_loop(0, (nch * rb) // 16, unpack_body, 0)


def _accumulate(Df, rb, x_hbm, msrc, mdl, rowbuf, acc, sem, nch):
    def chunk_body(j, _):
        base = j * rb
        cp = pltpu.async_copy(x_hbm.at[msrc.at[pl.ds(base, rb)]], rowbuf, sem)
        cp.wait()

        def group_body(g, _):
            dl16 = mdl[pl.ds(base + 16 * g, 16)]
            for lane in range(16):
                dl = dl16[lane]
                r = 16 * g + lane
                for c in range(Df // 16):
                    slc = pl.ds(16 * c, 16)
                    acc[dl, slc] = jnp.maximum(acc[dl, slc], rowbuf[r, slc])
            return 0

        lax.fori_loop(0, rb // 16, group_body, 0)
        return 0

    lax.fori_loop(0, nch, chunk_body, 0)


def _init_acc(Df, acc, neg):
    def init_body(r, _):
        for c in range(Df // 16):
            acc[r, pl.ds(16 * c, 16)] = neg
        return 0

    lax.fori_loop(0, PB + 1, init_body, 0)


def _seg_max_scan_body(Df, rb, x_hbm, src_hbm, dst_hbm,
                       out_hbm, lists_hbm, wcnts_hbm,
                       dbuf, sbuf, mpk, msrc, mdl, wcb, rowbuf, acc, sem):
    wid = lax.axis_index("s") * NC + lax.axis_index("c")
    lo = wid * PB
    lo16 = jnp.broadcast_to(lo, (16,))
    hi16 = lo16 + PB
    neg = jnp.full((16,), _NEG, jnp.float32)
    iota = lax.iota(jnp.int32, 16)
    pad_rows = wid * 16 + iota

    _init_acc(Df, acc, neg)

    def win_body(w, _):
        pltpu.sync_copy(dst_hbm.at[pl.ds(w * W, W)], dbuf)
        pltpu.sync_copy(src_hbm.at[pl.ds(w * W, W)], sbuf)

        def scan_body(v, cnt):
            sl = pl.ds(16 * v, 16)
            d16 = dbuf[sl]
            s16 = sbuf[sl]
            mask = (d16 >= lo16) & (d16 < hi16)
            val = s16 * 512 + (d16 - lo16)
            _, sv = plsc.sort_key_val(jnp.where(mask, 0, 1), val)
            mpk[pl.ds(cnt, 16)] = sv
            return cnt + plsc.all_reduce_population_count(mask)[0]

        cnt = lax.fori_loop(0, W // 16, scan_body, jnp.int32(0))
        wcb[pl.ds(16 * w, 16)] = jnp.broadcast_to(cnt, (16,))

        @pl.when(cnt > 0)
        def _process():
            pltpu.sync_copy(mpk.at[pl.ds(0, W)],
                            lists_hbm.at[pl.ds(wid * EPAD + w * W, W)])
            nch = (cnt + rb - 1) // rb
            _unpack_window(mpk, msrc, mdl, cnt, nch, rb, iota, pad_rows)
            _accumulate(Df, rb, x_hbm, msrc, mdl, rowbuf, acc, sem, nch)

        return 0

    lax.fori_loop(0, NWIN, win_body, 0)
    pltpu.sync_copy(wcb, wcnts_hbm.at[pl.ds(wid * WC, WC)])
    pltpu.sync_copy(acc.at[pl.ds(0, PB)], out_hbm.at[pl.ds(lo, PB)])


def _seg_max_replay_body(Df, rb, x_hbm, lists_hbm, wcnts_hbm, out_hbm,
                         mpk, msrc, mdl, wcb, rowbuf, acc, sem):
    wid = lax.axis_index("s") * NC + lax.axis_index("c")
    lo = wid * PB
    neg = jnp.full((16,), _NEG, jnp.float32)
    iota = lax.iota(jnp.int32, 16)
    pad_rows = wid * 16 + iota

    _init_acc(Df, acc, neg)
    pltpu.sync_copy(wcnts_hbm.at[pl.ds(wid * WC, WC)], wcb)

    def win_body(w, _):
        cnt = wcb[pl.ds(16 * w, 16)][0]

        @pl.when(cnt > 0)
        def _process():
            pltpu.sync_copy(lists_hbm.at[pl.ds(wid * EPAD + w * W, W)],
                            mpk.at[pl.ds(0, W)])
            nch = (cnt + rb - 1) // rb
            _unpack_window(mpk, msrc, mdl, cnt, nch, rb, iota, pad_rows)
            _accumulate(Df, rb, x_hbm, msrc, mdl, rowbuf, acc, sem, nch)

        return 0

    lax.fori_loop(0, NWIN, win_body, 0)
    pltpu.sync_copy(acc.at[pl.ds(0, PB)], out_hbm.at[pl.ds(lo, PB)])


_MESH = plsc.VectorSubcoreMesh(core_axis_name="c", subcore_axis_name="s")
_PARAMS = pltpu.CompilerParams(needs_layout_passes=False)


def _common_scratch(Df, rb):
    return [
        pltpu.VMEM((W + 16,), jnp.int32),
        pltpu.VMEM((W,), jnp.int32),
        pltpu.VMEM((W + 16,), jnp.int32),
        pltpu.VMEM((WC,), jnp.int32),
        pltpu.VMEM((rb, Df), jnp.float32),
        pltpu.VMEM((PB + 8, Df), jnp.float32),
        pltpu.SemaphoreType.DMA,
    ]


def _seg_max_scan(x, src, dst):
    n, Df = x.shape
    rb = 128
    kfn = pl.kernel(
        functools.partial(_seg_max_scan_body, Df, rb),
        mesh=_MESH,
        compiler_params=_PARAMS,
        out_type=(
            jax.ShapeDtypeStruct((N2, Df), jnp.float32),
            jax.ShapeDtypeStruct((NW * EPAD,), jnp.int32),
            jax.ShapeDtypeStruct((NW * WC,), jnp.int32),
        ),
        scratch_types=[
            pltpu.VMEM((W,), jnp.int32),
            pltpu.VMEM((W,), jnp.int32),
        ] + _common_scratch(Df, rb),
    )
    return kfn(x, src, dst)


def _seg_max_replay(x, lists, wcnts):
    n, Df = x.shape
    rb = 64
    kfn = pl.kernel(
        functools.partial(_seg_max_replay_body, Df, rb),
        mesh=_MESH,
        compiler_params=_PARAMS,
        out_type=jax.ShapeDtypeStruct((N2, Df), jnp.float32),
        scratch_types=_common_scratch(Df, rb),
    )
    return kfn(x, lists, wcnts)


def _mm_body(agg_ref, x_ref, wl_ref, b_ref, wr_ref, o_ref, *, relu):
    a = agg_ref[...]
    a = jnp.where(a <= _NEG, 0.0, a)
    o = (jnp.dot(a, wl_ref[...], preferred_element_type=jnp.float32)
         + b_ref[...][None, :]
         + jnp.dot(x_ref[...], wr_ref[...], preferred_element_type=jnp.float32))
    if relu:
        o = jnp.maximum(o, 0.0)
    o_ref[...] = o


def _mm(agg, x, wl, b, wr, relu):
    n, din = x.shape
    dout = wl.shape[1]
    rblk = 2000
    return pl.pallas_call(
        functools.partial(_mm_body, relu=relu),
        grid=(n // rblk,),
        in_specs=[
            pl.BlockSpec((rblk, din), lambda i: (i, 0)),
            pl.BlockSpec((rblk, din), lambda i: (i, 0)),
            pl.BlockSpec((din, dout), lambda i: (0, 0)),
            pl.BlockSpec((dout,), lambda i: (0,)),
            pl.BlockSpec((din, dout), lambda i: (0, 0)),
        ],
        out_specs=pl.BlockSpec((rblk, dout), lambda i: (i, 0)),
        out_shape=jax.ShapeDtypeStruct((n, dout), jnp.float32),
    )(agg, x, wl, b, wr)


def kernel(x, edge_index, W1_l, b1_l, W1_r, W2_l, b2_l, W2_r):
    pad = EPAD - E
    src = jnp.concatenate([edge_index[0], jnp.broadcast_to(edge_index[0, -1:], (pad,))])
    dst = jnp.concatenate([edge_index[1], jnp.broadcast_to(edge_index[1, -1:], (pad,))])
    agg1, lists, wcnts = _seg_max_scan(x, src, dst)
    h = _mm(agg1[:N], x, W1_l, b1_l, W1_r, relu=True)
    agg2 = _seg_max_replay(h, lists, wcnts)[:N]
    out = _mm(agg2, h, W2_l, b2_l, W2_r, relu=False)
    return out


# speedup vs baseline: 1.5724x; 1.5724x over previous
import functools

import jax
import jax.numpy as jnp
from jax import lax
from jax.experimental import pallas as pl
from jax.experimental.pallas import tpu as pltpu
from jax.experimental.pallas import tpu_sc as plsc

N = 10000
D = 128
H = 256
C = 64
E = 320000

NC = 2
NS = 16
NW = NC * NS

PB = 320
N2 = NW * PB
W = 2048
EPAD = ((E + W - 1) // W) * W
NWIN = EPAD // W
WC = 2560

_NEG = -3.0e38


def _unpack_window(mpk, msrc, mdl, cnt, nch, rb, iota, pad_rows):
    def unpack_body(u, _):
        gidx = 16 * u + iota
        p = mpk[pl.ds(16 * u, 16)]
        valid = gidx < cnt
        msrc[pl.ds(16 * u, 16)] = jnp.where(
            valid, lax.shift_right_logical(p, 9), pad_rows)
        mdl[pl.ds(16 * u, 16)] = jnp.where(valid, p & 511, PB)
        return 0

    lax.fori_loop(0, (nch * rb) // 16, unpack_body, 0)


def _accumulate(Df, rb, x_hbm, msrc, mdl, rowbuf, acc, sem, nch):
    def chunk_body(j, _):
        base = j * rb
        cp = pltpu.async_copy(x_hbm.at[msrc.at[pl.ds(base, rb)]], rowbuf, sem)
        cp.wait()

        def group_body(g, _):
            dl16 = mdl[pl.ds(base + 16 * g, 16)]
            dls = [dl16[lane] for lane in range(16)]
            for lane in range(16):
                dl = dls[lane]
                r = 16 * g + lane
                avs = [acc[dl, pl.ds(16 * c, 16)] for c in range(Df // 16)]
                rvs = [rowbuf[r, pl.ds(16 * c, 16)] for c in range(Df // 16)]
                mvs = [jnp.maximum(a, b) for a, b in zip(avs, rvs)]
                for c in range(Df // 16):
                    acc[dl, pl.ds(16 * c, 16)] = mvs[c]
            return 0

        lax.fori_loop(0, rb // 16, group_body, 0)
        return 0

    lax.fori_loop(0, nch, chunk_body, 0)


def _init_acc(Df, acc, neg):
    def init_body(r, _):
        for c in range(Df // 16):
            acc[r, pl.ds(16 * c, 16)] = neg
        return 0

    lax.fori_loop(0, PB + 1, init_body, 0)


def _seg_max_scan_body(Df, rb, x_hbm, src_hbm, dst_hbm,
                       out_hbm, lists_hbm, wcnts_hbm,
                       dbuf, sbuf, mpk, msrc, mdl, wcb, rowbuf, acc, sem):
    wid = lax.axis_index("s") * NC + lax.axis_index("c")
    lo = wid * PB
    lo16 = jnp.broadcast_to(lo, (16,))
    hi16 = lo16 + PB
    neg = jnp.full((16,), _NEG, jnp.float32)
    iota = lax.iota(jnp.int32, 16)
    pad_rows = wid * 16 + iota

    _init_acc(Df, acc, neg)

    def win_body(w, _):
        pltpu.sync_copy(dst_hbm.at[pl.ds(w * W, W)], dbuf)
        pltpu.sync_copy(src_hbm.at[pl.ds(w * W, W)], sbuf)

        def scan_body(v, cnt):
            sl = pl.ds(16 * v, 16)
            d16 = dbuf[sl]
            s16 = sbuf[sl]
            mask = (d16 >= lo16) & (d16 < hi16)
            val = s16 * 512 + (d16 - lo16)
            _, sv = plsc.sort_key_val(jnp.where(mask, 0, 1), val)
            mpk[pl.ds(cnt, 16)] = sv
            return cnt + plsc.all_reduce_population_count(mask)[0]

        cnt = lax.fori_loop(0, W // 16, scan_body, jnp.int32(0))
        wcb[pl.ds(16 * w, 16)] = jnp.broadcast_to(cnt, (16,))

        @pl.when(cnt > 0)
        def _process():
            pltpu.sync_copy(mpk.at[pl.ds(0, W)],
                            lists_hbm.at[pl.ds(wid * EPAD + w * W, W)])
            nch = (cnt + rb - 1) // rb
            _unpack_window(mpk, msrc, mdl, cnt, nch, rb, iota, pad_rows)
            _accumulate(Df, rb, x_hbm, msrc, mdl, rowbuf, acc, sem, nch)

        return 0

    lax.fori_loop(0, NWIN, win_body, 0)
    pltpu.sync_copy(wcb, wcnts_hbm.at[pl.ds(wid * WC, WC)])
    pltpu.sync_copy(acc.at[pl.ds(0, PB)], out_hbm.at[pl.ds(lo, PB)])


def _seg_max_replay_body(Df, rb, x_hbm, lists_hbm, wcnts_hbm, out_hbm,
                         mpk, msrc, mdl, wcb, rowbuf, acc, sem):
    wid = lax.axis_index("s") * NC + lax.axis_index("c")
    lo = wid * PB
    neg = jnp.full((16,), _NEG, jnp.float32)
    iota = lax.iota(jnp.int32, 16)
    pad_rows = wid * 16 + iota

    _init_acc(Df, acc, neg)
    pltpu.sync_copy(wcnts_hbm.at[pl.ds(wid * WC, WC)], wcb)

    def win_body(w, _):
        cnt = wcb[pl.ds(16 * w, 16)][0]

        @pl.when(cnt > 0)
        def _process():
            pltpu.sync_copy(lists_hbm.at[pl.ds(wid * EPAD + w * W, W)],
                            mpk.at[pl.ds(0, W)])
            nch = (cnt + rb - 1) // rb
            _unpack_window(mpk, msrc, mdl, cnt, nch, rb, iota, pad_rows)
            _accumulate(Df, rb, x_hbm, msrc, mdl, rowbuf, acc, sem, nch)

        return 0

    lax.fori_loop(0, NWIN, win_body, 0)
    pltpu.sync_copy(acc.at[pl.ds(0, PB)], out_hbm.at[pl.ds(lo, PB)])


_MESH = plsc.VectorSubcoreMesh(core_axis_name="c", subcore_axis_name="s")
_PARAMS = pltpu.CompilerParams(needs_layout_passes=False)


def _common_scratch(Df, rb):
    return [
        pltpu.VMEM((W + 16,), jnp.int32),
        pltpu.VMEM((W,), jnp.int32),
        pltpu.VMEM((W + 16,), jnp.int32),
        pltpu.VMEM((WC,), jnp.int32),
        pltpu.VMEM((rb, Df), jnp.float32),
        pltpu.VMEM((PB + 8, Df), jnp.float32),
        pltpu.SemaphoreType.DMA,
    ]


def _seg_max_scan(x, src, dst):
    n, Df = x.shape
    rb = 128
    kfn = pl.kernel(
        functools.partial(_seg_max_scan_body, Df, rb),
        mesh=_MESH,
        compiler_params=_PARAMS,
        out_type=(
            jax.ShapeDtypeStruct((N2, Df), jnp.float32),
            jax.ShapeDtypeStruct((NW * EPAD,), jnp.int32),
            jax.ShapeDtypeStruct((NW * WC,), jnp.int32),
        ),
        scratch_types=[
            pltpu.VMEM((W,), jnp.int32),
            pltpu.VMEM((W,), jnp.int32),
        ] + _common_scratch(Df, rb),
    )
    return kfn(x, src, dst)


def _seg_max_replay(x, lists, wcnts):
    n, Df = x.shape
    rb = 64
    kfn = pl.kernel(
        functools.partial(_seg_max_replay_body, Df, rb),
        mesh=_MESH,
        compiler_params=_PARAMS,
        out_type=jax.ShapeDtypeStruct((N2, Df), jnp.float32),
        scratch_types=_common_scratch(Df, rb),
    )
    return kfn(x, lists, wcnts)


def _mm_body(agg_ref, x_ref, wl_ref, b_ref, wr_ref, o_ref, *, relu):
    a = agg_ref[...]
    a = jnp.where(a <= _NEG, 0.0, a)
    o = (jnp.dot(a, wl_ref[...], preferred_element_type=jnp.float32)
         + b_ref[...][None, :]
         + jnp.dot(x_ref[...], wr_ref[...], preferred_element_type=jnp.float32))
    if relu:
        o = jnp.maximum(o, 0.0)
    o_ref[...] = o


def _mm(agg, x, wl, b, wr, relu):
    n, din = x.shape
    dout = wl.shape[1]
    rblk = 2000
    return pl.pallas_call(
        functools.partial(_mm_body, relu=relu),
        grid=(n // rblk,),
        in_specs=[
            pl.BlockSpec((rblk, din), lambda i: (i, 0)),
            pl.BlockSpec((rblk, din), lambda i: (i, 0)),
            pl.BlockSpec((din, dout), lambda i: (0, 0)),
            pl.BlockSpec((dout,), lambda i: (0,)),
            pl.BlockSpec((din, dout), lambda i: (0, 0)),
        ],
        out_specs=pl.BlockSpec((rblk, dout), lambda i: (i, 0)),
        out_shape=jax.ShapeDtypeStruct((n, dout), jnp.float32),
    )(agg, x, wl, b, wr)


def kernel(x, edge_index, W1_l, b1_l, W1_r, W2_l, b2_l, W2_r):
    pad = EPAD - E
    src = jnp.concatenate([edge_index[0], jnp.broadcast_to(edge_index[0, -1:], (pad,))])
    dst = jnp.concatenate([edge_index[1], jnp.broadcast_to(edge_index[1, -1:], (pad,))])
    agg1, lists, wcnts = _seg_max_scan(x, src, dst)
    h = _mm(agg1[:N], x, W1_l, b1_l, W1_r, relu=True)
    agg2 = _seg_max_replay(h, lists, wcnts)[:N]
    out = _mm(agg2, h, W2_l, b2_l, W2_r, relu=False)
    return out


# speedup vs baseline: 2.0708x; 1.3170x over previous
import functools

import jax
import jax.numpy as jnp
from jax import lax
from jax.experimental import pallas as pl
from jax.experimental.pallas import tpu as pltpu
from jax.experimental.pallas import tpu_sc as plsc

N = 10000
D = 128
H = 256
C = 64
E = 320000

NC = 2
NS = 16
NW = NC * NS

PB = 320
N2 = NW * PB
W = 2048
EPAD = ((E + 2 * W - 1) // (2 * W)) * (2 * W)
NWIN = EPAD // W
LCAP = ((EPAD + 16 * NWIN + 64 + 127) // 128) * 128
SW = 2048

_NEG = -3.0e38


def _gather_accumulate(Df, rb, x_hbm, msrc, mdl, rowA, rowB, gsA, gsB, acc, nch):
    def start(j, buf, sem):
        pltpu.async_copy(x_hbm.at[msrc.at[pl.ds(j * rb, rb)]], buf, sem)

    def wait(buf, sem):
        pltpu.make_async_copy(x_hbm.at[msrc.at[pl.ds(0, rb)]], buf, sem).wait()

    def process(j, buf):
        base = j * rb

        def group_body(g, _):
            dl16 = mdl[pl.ds(base + 16 * g, 16)]
            dls = [dl16[lane] for lane in range(16)]
            for lane in range(16):
                dl = dls[lane]
                r = 16 * g + lane
                avs = [acc[dl, pl.ds(16 * c, 16)] for c in range(Df // 16)]
                rvs = [buf[r, pl.ds(16 * c, 16)] for c in range(Df // 16)]
                mvs = [jnp.maximum(a, b) for a, b in zip(avs, rvs)]
                for c in range(Df // 16):
                    acc[dl, pl.ds(16 * c, 16)] = mvs[c]
            return 0

        lax.fori_loop(0, rb // 16, group_body, 0)

    @pl.when(nch > 0)
    def _prime():
        start(0, rowA, gsA)

    def pair_body(jp, _):
        j0 = 2 * jp
        j1 = j0 + 1

        @pl.when(j1 < nch)
        def _():
            start(j1, rowB, gsB)

        wait(rowA, gsA)
        process(j0, rowA)

        @pl.when(j1 < nch)
        def _():
            @pl.when(j1 + 1 < nch)
            def _():
                start(j1 + 1, rowA, gsA)

            wait(rowB, gsB)
            process(j1, rowB)

        return 0

    lax.fori_loop(0, (nch + 1) // 2, pair_body, 0)


def _init_acc(Df, acc, neg):
    def init_body(r, _):
        for c in range(Df // 16):
            acc[r, pl.ds(16 * c, 16)] = neg
        return 0

    lax.fori_loop(0, PB + 1, init_body, 0)


def _seg_max_scan_body(Df, rb, x_hbm, src_hbm, dst_hbm,
                       out_hbm, lists_hbm, tot_hbm,
                       dbA, sbA, dbB, sbB, mpk, msrc, mdl, wcb,
                       rowA, rowB, acc, ssA, ssB, gsA, gsB, wsem):
    wid = lax.axis_index("s") * NC + lax.axis_index("c")
    lo = wid * PB
    lbase = wid * LCAP
    lo16 = jnp.broadcast_to(lo, (16,))
    hi16 = lo16 + PB
    neg = jnp.full((16,), _NEG, jnp.float32)
    iota = lax.iota(jnp.int32, 16)
    pad_rows = wid * 16 + iota
    pad_pk = pad_rows * 512 + PB

    _init_acc(Df, acc, neg)

    def stage_start(w, db, sb, sem):
        pltpu.async_copy(dst_hbm.at[pl.ds(w * W, W)], db, sem)
        pltpu.async_copy(src_hbm.at[pl.ds(w * W, W)], sb, sem)

    def stage_wait(db, sb, sem):
        pltpu.make_async_copy(dst_hbm.at[pl.ds(0, W)], db, sem).wait()
        pltpu.make_async_copy(src_hbm.at[pl.ds(0, W)], sb, sem).wait()

    def do_window(w, db, sb, total):
        def scan_body(v, cnt):
            sl = pl.ds(16 * v, 16)
            d16 = db[sl]
            s16 = sb[sl]
            mask = (d16 >= lo16) & (d16 < hi16)
            val = s16 * 512 + (d16 - lo16)
            _, sv = plsc.sort_key_val(jnp.where(mask, 0, 1), val)
            mpk[pl.ds(cnt, 16)] = sv
            return cnt + plsc.all_reduce_population_count(mask)[0]

        cnt = lax.fori_loop(0, W // 16, scan_body, jnp.int32(0))
        mpk[pl.ds(cnt, 16)] = pad_pk
        nq = (cnt + 15) // 16
        tot16 = pl.multiple_of(total, 16)

        @pl.when(cnt > 0)
        def _process():
            def wr_body(q, _):
                pltpu.async_copy(
                    mpk.at[pl.ds(16 * q, 16)],
                    lists_hbm.at[pl.ds(lbase + tot16 + 16 * q, 16)], wsem)
                return 0

            lax.fori_loop(0, nq, wr_body, 0)

            nch = (cnt + rb - 1) // rb
            lim = cnt + 16

            def unpack_body(u, _):
                gidx = 16 * u + iota
                p = mpk[pl.ds(16 * u, 16)]
                valid = gidx < lim
                msrc[pl.ds(16 * u, 16)] = jnp.where(
                    valid, lax.shift_right_logical(p, 9), pad_rows)
                mdl[pl.ds(16 * u, 16)] = jnp.where(valid, p & 511, PB)
                return 0

            lax.fori_loop(0, (nch * rb) // 16, unpack_body, 0)
            _gather_accumulate(Df, rb, x_hbm, msrc, mdl,
                               rowA, rowB, gsA, gsB, acc, nch)

            def wrw_body(q, _):
                pltpu.make_async_copy(
                    mpk.at[pl.ds(0, 16)],
                    lists_hbm.at[pl.ds(lbase, 16)], wsem).wait()
                return 0

            lax.fori_loop(0, nq, wrw_body, 0)

        return total + nq * 16

    stage_start(0, dbA, sbA, ssA)

    def pair_body(jp, total):
        w0 = 2 * jp
        stage_start(w0 + 1, dbB, sbB, ssB)
        stage_wait(dbA, sbA, ssA)
        total = do_window(w0, dbA, sbA, total)

        @pl.when(jp + 1 < NWIN // 2)
        def _():
            stage_start(w0 + 2, dbA, sbA, ssA)

        stage_wait(dbB, sbB, ssB)
        total = do_window(w0 + 1, dbB, sbB, total)
        return total

    total = lax.fori_loop(0, NWIN // 2, pair_body, jnp.int32(0))

    mpk[pl.ds(0, 16)] = pad_pk
    tot16 = pl.multiple_of(total, 16)

    def fin_body(q, _):
        pltpu.async_copy(mpk.at[pl.ds(0, 16)],
                         lists_hbm.at[pl.ds(lbase + tot16 + 16 * q, 16)], wsem)
        return 0

    lax.fori_loop(0, 4, fin_body, 0)

    def fin_wait(q, _):
        pltpu.make_async_copy(mpk.at[pl.ds(0, 16)],
                              lists_hbm.at[pl.ds(lbase, 16)], wsem).wait()
        return 0

    lax.fori_loop(0, 4, fin_wait, 0)

    wcb[pl.ds(0, 16)] = jnp.broadcast_to(total, (16,))
    pltpu.sync_copy(wcb, tot_hbm.at[pl.ds(wid * 128, 128)])
    pltpu.sync_copy(acc.at[pl.ds(0, PB)], out_hbm.at[pl.ds(lo, PB)])


def _seg_max_replay_body(Df, rb, x_hbm, lists_hbm, tot_hbm, out_hbm,
                         mpA, mpB, msrc, mdl, wcb,
                         rowA, rowB, acc, ssA, ssB, gsA, gsB):
    wid = lax.axis_index("s") * NC + lax.axis_index("c")
    lo = wid * PB
    lbase = wid * LCAP
    neg = jnp.full((16,), _NEG, jnp.float32)
    iota = lax.iota(jnp.int32, 16)

    _init_acc(Df, acc, neg)
    pltpu.sync_copy(tot_hbm.at[pl.ds(wid * 128, 128)], wcb)
    total = wcb[pl.ds(0, 16)][0]
    lt = pl.multiple_of(((total + 63) // 64) * 64, 64)
    nchk = (lt + SW - 1) // SW

    def stage_start(c, mp, sem):
        pltpu.async_copy(lists_hbm.at[pl.ds(lbase + c * SW, SW)], mp, sem)

    def stage_wait(mp, sem):
        pltpu.make_async_copy(lists_hbm.at[pl.ds(lbase, SW)], mp, sem).wait()

    def proc_chunk(c, mp):
        nb = jnp.minimum(lt - c * SW, SW) // rb

        def unpack_body(u, _):
            p = mp[pl.ds(16 * u, 16)]
            msrc[pl.ds(16 * u, 16)] = lax.shift_right_logical(p, 9)
            mdl[pl.ds(16 * u, 16)] = p & 511
            return 0

        lax.fori_loop(0, (nb * rb) // 16, unpack_body, 0)
        _gather_accumulate(Df, rb, x_hbm, msrc, mdl,
                           rowA, rowB, gsA, gsB, acc, nb)

    @pl.when(nchk > 0)
    def _prime():
        stage_start(0, mpA, ssA)

    def pair_body(cp, _):
        c0 = 2 * cp
        c1 = c0 + 1

        @pl.when(c1 < nchk)
        def _():
            stage_start(c1, mpB, ssB)

        stage_wait(mpA, ssA)
        proc_chunk(c0, mpA)

        @pl.when(c1 < nchk)
        def _():
            @pl.when(c1 + 1 < nchk)
            def _():
                stage_start(c1 + 1, mpA, ssA)

            stage_wait(mpB, ssB)
            proc_chunk(c1, mpB)

        return 0

    lax.fori_loop(0, (nchk + 1) // 2, pair_body, 0)
    pltpu.sync_copy(acc.at[pl.ds(0, PB)], out_hbm.at[pl.ds(lo, PB)])


_MESH = plsc.VectorSubcoreMesh(core_axis_name="c", subcore_axis_name="s")
_PARAMS = pltpu.CompilerParams(needs_layout_passes=False)


def _seg_max_scan(x, src, dst):
    n, Df = x.shape
    rb = 128
    kfn = pl.kernel(
        functools.partial(_seg_max_scan_body, Df, rb),
        mesh=_MESH,
        compiler_params=_PARAMS,
        out_type=(
            jax.ShapeDtypeStruct((N2, Df), jnp.float32),
            jax.ShapeDtypeStruct((NW * LCAP + 2048,), jnp.int32),
            jax.ShapeDtypeStruct((NW * 128,), jnp.int32),
        ),
        scratch_types=[
            pltpu.VMEM((W,), jnp.int32),
            pltpu.VMEM((W,), jnp.int32),
            pltpu.VMEM((W,), jnp.int32),
            pltpu.VMEM((W,), jnp.int32),
            pltpu.VMEM((W + 16,), jnp.int32),
            pltpu.VMEM((W,), jnp.int32),
            pltpu.VMEM((W + 16,), jnp.int32),
            pltpu.VMEM((128,), jnp.int32),
            pltpu.VMEM((rb, Df), jnp.float32),
            pltpu.VMEM((rb, Df), jnp.float32),
            pltpu.VMEM((PB + 8, Df), jnp.float32),
            pltpu.SemaphoreType.DMA,
            pltpu.SemaphoreType.DMA,
            pltpu.SemaphoreType.DMA,
            pltpu.SemaphoreType.DMA,
            pltpu.SemaphoreType.DMA,
        ],
    )
    return kfn(x, src, dst)


def _seg_max_replay(x, lists, tots):
    n, Df = x.shape
    rb = 64
    kfn = pl.kernel(
        functools.partial(_seg_max_replay_body, Df, rb),
        mesh=_MESH,
        compiler_params=_PARAMS,
        out_type=jax.ShapeDtypeStruct((N2, Df), jnp.float32),
        scratch_types=[
            pltpu.VMEM((SW,), jnp.int32),
            pltpu.VMEM((SW,), jnp.int32),
            pltpu.VMEM((SW,), jnp.int32),
            pltpu.VMEM((SW + 16,), jnp.int32),
            pltpu.VMEM((128,), jnp.int32),
            pltpu.VMEM((rb, Df), jnp.float32),
            pltpu.VMEM((rb, Df), jnp.float32),
            pltpu.VMEM((PB + 8, Df), jnp.float32),
            pltpu.SemaphoreType.DMA,
            pltpu.SemaphoreType.DMA,
            pltpu.SemaphoreType.DMA,
            pltpu.SemaphoreType.DMA,
        ],
    )
    return kfn(x, lists, tots)


def _mm_body(agg_ref, x_ref, wl_ref, b_ref, wr_ref, o_ref, *, relu):
    a = agg_ref[...]
    a = jnp.where(a <= _NEG, 0.0, a)
    o = (jnp.dot(a, wl_ref[...], preferred_element_type=jnp.float32)
         + b_ref[...][None, :]
         + jnp.dot(x_ref[...], wr_ref[...], preferred_element_type=jnp.float32))
    if relu:
        o = jnp.maximum(o, 0.0)
    o_ref[...] = o


def _mm(agg, x, wl, b, wr, relu):
    n, din = x.shape
    dout = wl.shape[1]
    rblk = 2000
    return pl.pallas_call(
        functools.partial(_mm_body, relu=relu),
        grid=(n // rblk,),
        in_specs=[
            pl.BlockSpec((rblk, din), lambda i: (i, 0)),
            pl.BlockSpec((rblk, din), lambda i: (i, 0)),
            pl.BlockSpec((din, dout), lambda i: (0, 0)),
            pl.BlockSpec((dout,), lambda i: (0,)),
            pl.BlockSpec((din, dout), lambda i: (0, 0)),
        ],
        out_specs=pl.BlockSpec((rblk, dout), lambda i: (i, 0)),
        out_shape=jax.ShapeDtypeStruct((n, dout), jnp.float32),
    )(agg, x, wl, b, wr)


def kernel(x, edge_index, W1_l, b1_l, W1_r, W2_l, b2_l, W2_r):
    pad = EPAD - E
    src = jnp.concatenate([edge_index[0], jnp.broadcast_to(edge_index[0, -1:], (pad,))])
    dst = jnp.concatenate([edge_index[1], jnp.broadcast_to(edge_index[1, -1:], (pad,))])
    agg1, lists, tots = _seg_max_scan(x, src, dst)
    h = _mm(agg1[:N], x, W1_l, b1_l, W1_r, relu=True)
    agg2 = _seg_max_replay(h, lists, tots)[:N]
    out = _mm(agg2, h, W2_l, b2_l, W2_r, relu=False)
    return out


# speedup vs baseline: 2.5536x; 1.2332x over previous
import functools

import jax
import jax.numpy as jnp
from jax import lax
from jax.experimental import pallas as pl
from jax.experimental.pallas import tpu as pltpu
from jax.experimental.pallas import tpu_sc as plsc

N = 10000
D = 128
H = 256
C = 64
E = 320000

NC = 2
NS = 16
NW = NC * NS

PB = 320
N2 = NW * PB
W = 2048
EPAD = ((E + 2 * W - 1) // (2 * W)) * (2 * W)
NWIN = EPAD // W
LCAP = ((EPAD + 16 * NWIN + 128 + 127) // 128) * 128
SW = 2048

_NEG = -3.0e38


def _gather_accumulate(Df, rb, x_hbm, msrc, mdl, rowA, rowB, gsA, gsB, acc, nch):
    def start(j, buf, sem):
        pltpu.async_copy(x_hbm.at[msrc.at[pl.ds(j * rb, rb)]], buf, sem)

    def wait(buf, sem):
        pltpu.make_async_copy(x_hbm.at[msrc.at[pl.ds(0, rb)]], buf, sem).wait()

    def process(j, buf):
        base = j * rb

        def group_body(g, _):
            dl16 = mdl[pl.ds(base + 16 * g, 16)]
            dls = [dl16[lane] for lane in range(16)]
            for lane in range(16):
                dl = dls[lane]
                r = 16 * g + lane
                avs = [acc[dl, pl.ds(16 * c, 16)] for c in range(Df // 16)]
                rvs = [buf[r, pl.ds(16 * c, 16)] for c in range(Df // 16)]
                mvs = [jnp.maximum(a, b) for a, b in zip(avs, rvs)]
                for c in range(Df // 16):
                    acc[dl, pl.ds(16 * c, 16)] = mvs[c]
            return 0

        lax.fori_loop(0, rb // 16, group_body, 0)

    @pl.when(nch > 0)
    def _prime():
        start(0, rowA, gsA)

    def pair_body(jp, _):
        j0 = 2 * jp
        j1 = j0 + 1

        @pl.when(j1 < nch)
        def _():
            start(j1, rowB, gsB)

        wait(rowA, gsA)
        process(j0, rowA)

        @pl.when(j1 < nch)
        def _():
            @pl.when(j1 + 1 < nch)
            def _():
                start(j1 + 1, rowA, gsA)

            wait(rowB, gsB)
            process(j1, rowB)

        return 0

    lax.fori_loop(0, (nch + 1) // 2, pair_body, 0)


def _replay_core(Df, rb, x_hbm, lists_hbm, lbase, total,
                 mpA, mpB, msrc, mdl, rowA, rowB, acc, ssA, ssB, gsA, gsB):
    lt = pl.multiple_of(((total + rb - 1) // rb) * rb, rb)
    nchk = (lt + SW - 1) // SW

    def stage_start(c, mp, sem):
        pltpu.async_copy(lists_hbm.at[pl.ds(lbase + c * SW, SW)], mp, sem)

    def stage_wait(mp, sem):
        pltpu.make_async_copy(lists_hbm.at[pl.ds(lbase, SW)], mp, sem).wait()

    def proc_chunk(c, mp):
        nb = jnp.minimum(lt - c * SW, SW) // rb

        def unpack_body(u, _):
            p = mp[pl.ds(16 * u, 16)]
            msrc[pl.ds(16 * u, 16)] = lax.shift_right_logical(p, 9)
            mdl[pl.ds(16 * u, 16)] = p & 511
            return 0

        lax.fori_loop(0, (nb * rb) // 16, unpack_body, 0)
        _gather_accumulate(Df, rb, x_hbm, msrc, mdl,
                           rowA, rowB, gsA, gsB, acc, nb)

    @pl.when(nchk > 0)
    def _prime():
        stage_start(0, mpA, ssA)

    def pair_body(cp, _):
        c0 = 2 * cp
        c1 = c0 + 1

        @pl.when(c1 < nchk)
        def _():
            stage_start(c1, mpB, ssB)

        stage_wait(mpA, ssA)
        proc_chunk(c0, mpA)

        @pl.when(c1 < nchk)
        def _():
            @pl.when(c1 + 1 < nchk)
            def _():
                stage_start(c1 + 1, mpA, ssA)

            stage_wait(mpB, ssB)
            proc_chunk(c1, mpB)

        return 0

    lax.fori_loop(0, (nchk + 1) // 2, pair_body, 0)


def _init_acc(Df, acc, neg):
    def init_body(r, _):
        for c in range(Df // 16):
            acc[r, pl.ds(16 * c, 16)] = neg
        return 0

    lax.fori_loop(0, PB + 1, init_body, 0)


def _seg_max_scan_body(Df, rb, x_hbm, src_hbm, dst_hbm,
                       out_hbm, lists_hbm, tot_hbm,
                       dbA, sbA, dbB, sbB, mpkA, mpkB, mpA, mpB,
                       msrc, mdl, wcb, rowA, rowB, acc,
                       ssA, ssB, gsA, gsB, wsA, wsB):
    wid = lax.axis_index("s") * NC + lax.axis_index("c")
    lo = wid * PB
    lbase = wid * LCAP
    lo16 = jnp.broadcast_to(lo, (16,))
    hi16 = lo16 + PB
    neg = jnp.full((16,), _NEG, jnp.float32)
    iota = lax.iota(jnp.int32, 16)
    pad_rows = wid * 16 + iota
    pad_pk = pad_rows * 512 + PB

    _init_acc(Df, acc, neg)

    def stage_start(w, db, sb, sem):
        pltpu.async_copy(dst_hbm.at[pl.ds(w * W, W)], db, sem)
        pltpu.async_copy(src_hbm.at[pl.ds(w * W, W)], sb, sem)

    def stage_wait(db, sb, sem):
        pltpu.make_async_copy(dst_hbm.at[pl.ds(0, W)], db, sem).wait()
        pltpu.make_async_copy(src_hbm.at[pl.ds(0, W)], sb, sem).wait()

    def drain(mpk, wsem, pend):
        def wrw_body(q, _):
            pltpu.make_async_copy(mpk.at[pl.ds(0, 16)],
                                  lists_hbm.at[pl.ds(lbase, 16)], wsem).wait()
            return 0

        lax.fori_loop(0, pend, wrw_body, 0)

    def scan_window(w, db, sb, mpk, wsem, total, pend):
        drain(mpk, wsem, pend)

        def scan_body(v, cnt):
            sl = pl.ds(16 * v, 16)
            d16 = db[sl]
            s16 = sb[sl]
            mask = (d16 >= lo16) & (d16 < hi16)
            val = s16 * 512 + (d16 - lo16)
            _, sv = plsc.sort_key_val(jnp.where(mask, 0, 1), val)
            mpk[pl.ds(cnt, 16)] = sv
            return cnt + plsc.all_reduce_population_count(mask)[0]

        cnt = lax.fori_loop(0, W // 16, scan_body, jnp.int32(0))
        mpk[pl.ds(cnt, 16)] = pad_pk
        nq = (cnt + 15) // 16
        tot16 = pl.multiple_of(total, 16)

        @pl.when(cnt > 0)
        def _emit():
            def wr_body(q, _):
                pltpu.async_copy(
                    mpk.at[pl.ds(16 * q, 16)],
                    lists_hbm.at[pl.ds(lbase + tot16 + 16 * q, 16)], wsem)
                return 0

            lax.fori_loop(0, nq, wr_body, 0)

        return total + nq * 16, nq

    stage_start(0, dbA, sbA, ssA)

    def pair_body(jp, carry):
        total, pendA, pendB = carry
        w0 = 2 * jp
        stage_start(w0 + 1, dbB, sbB, ssB)
        stage_wait(dbA, sbA, ssA)
        total, pendA = scan_window(w0, dbA, sbA, mpkA, wsA, total, pendA)

        @pl.when(jp + 1 < NWIN // 2)
        def _():
            stage_start(w0 + 2, dbA, sbA, ssA)

        stage_wait(dbB, sbB, ssB)
        total, pendB = scan_window(w0 + 1, dbB, sbB, mpkB, wsB, total, pendB)
        return total, pendA, pendB

    total, pendA, pendB = lax.fori_loop(
        0, NWIN // 2, pair_body,
        (jnp.int32(0), jnp.int32(0), jnp.int32(0)))
    drain(mpkA, wsA, pendA)
    drain(mpkB, wsB, pendB)

    mpkA[pl.ds(0, 16)] = pad_pk
    tot16 = pl.multiple_of(total, 16)

    def fin_body(q, _):
        pltpu.async_copy(mpkA.at[pl.ds(0, 16)],
                         lists_hbm.at[pl.ds(lbase + tot16 + 16 * q, 16)], wsA)
        return 0

    lax.fori_loop(0, 8, fin_body, 0)
    drain(mpkA, wsA, 8)

    wcb[pl.ds(0, 16)] = jnp.broadcast_to(total, (16,))
    pltpu.sync_copy(wcb, tot_hbm.at[pl.ds(wid * 128, 128)])

    _replay_core(Df, rb, x_hbm, lists_hbm, lbase, total,
                 mpA, mpB, msrc, mdl, rowA, rowB, acc, ssA, ssB, gsA, gsB)
    pltpu.sync_copy(acc.at[pl.ds(0, PB)], out_hbm.at[pl.ds(lo, PB)])


def _seg_max_replay_body(Df, rb, x_hbm, lists_hbm, tot_hbm, out_hbm,
                         mpA, mpB, msrc, mdl, wcb,
                         rowA, rowB, acc, ssA, ssB, gsA, gsB):
    wid = lax.axis_index("s") * NC + lax.axis_index("c")
    lo = wid * PB
    lbase = wid * LCAP
    neg = jnp.full((16,), _NEG, jnp.float32)

    _init_acc(Df, acc, neg)
    pltpu.sync_copy(tot_hbm.at[pl.ds(wid * 128, 128)], wcb)
    total = wcb[pl.ds(0, 16)][0]
    _replay_core(Df, rb, x_hbm, lists_hbm, lbase, total,
                 mpA, mpB, msrc, mdl, rowA, rowB, acc, ssA, ssB, gsA, gsB)
    pltpu.sync_copy(acc.at[pl.ds(0, PB)], out_hbm.at[pl.ds(lo, PB)])


_MESH = plsc.VectorSubcoreMesh(core_axis_name="c", subcore_axis_name="s")
_PARAMS = pltpu.CompilerParams(needs_layout_passes=False)


def _seg_max_scan(x, src, dst):
    n, Df = x.shape
    rb = 128
    kfn = pl.kernel(
        functools.partial(_seg_max_scan_body, Df, rb),
        mesh=_MESH,
        compiler_params=_PARAMS,
        out_type=(
            jax.ShapeDtypeStruct((N2, Df), jnp.float32),
            jax.ShapeDtypeStruct((NW * LCAP + 2048,), jnp.int32),
            jax.ShapeDtypeStruct((NW * 128,), jnp.int32),
        ),
        scratch_types=[
            pltpu.VMEM((W,), jnp.int32),
            pltpu.VMEM((W,), jnp.int32),
            pltpu.VMEM((W,), jnp.int32),
            pltpu.VMEM((W,), jnp.int32),
            pltpu.VMEM((W + 16,), jnp.int32),
            pltpu.VMEM((W + 16,), jnp.int32),
            pltpu.VMEM((SW,), jnp.int32),
            pltpu.VMEM((SW,), jnp.int32),
            pltpu.VMEM((SW,), jnp.int32),
            pltpu.VMEM((SW + 16,), jnp.int32),
            pltpu.VMEM((128,), jnp.int32),
            pltpu.VMEM((rb, Df), jnp.float32),
            pltpu.VMEM((rb, Df), jnp.float32),
            pltpu.VMEM((PB + 8, Df), jnp.float32),
            pltpu.SemaphoreType.DMA,
            pltpu.SemaphoreType.DMA,
            pltpu.SemaphoreType.DMA,
            pltpu.SemaphoreType.DMA,
            pltpu.SemaphoreType.DMA,
            pltpu.SemaphoreType.DMA,
        ],
    )
    return kfn(x, src, dst)


def _seg_max_replay(x, lists, tots):
    n, Df = x.shape
    rb = 64
    kfn = pl.kernel(
        functools.partial(_seg_max_replay_body, Df, rb),
        mesh=_MESH,
        compiler_params=_PARAMS,
        out_type=jax.ShapeDtypeStruct((N2, Df), jnp.float32),
        scratch_types=[
            pltpu.VMEM((SW,), jnp.int32),
            pltpu.VMEM((SW,), jnp.int32),
            pltpu.VMEM((SW,), jnp.int32),
            pltpu.VMEM((SW + 16,), jnp.int32),
            pltpu.VMEM((128,), jnp.int32),
            pltpu.VMEM((rb, Df), jnp.float32),
            pltpu.VMEM((rb, Df), jnp.float32),
            pltpu.VMEM((PB + 8, Df), jnp.float32),
            pltpu.SemaphoreType.DMA,
            pltpu.SemaphoreType.DMA,
            pltpu.SemaphoreType.DMA,
            pltpu.SemaphoreType.DMA,
        ],
    )
    return kfn(x, lists, tots)


def _mm_body(agg_ref, x_ref, wl_ref, b_ref, wr_ref, o_ref, *, relu):
    a = agg_ref[...]
    a = jnp.where(a <= _NEG, 0.0, a)
    o = (jnp.dot(a, wl_ref[...], preferred_element_type=jnp.float32)
         + b_ref[...][None, :]
         + jnp.dot(x_ref[...], wr_ref[...], preferred_element_type=jnp.float32))
    if relu:
        o = jnp.maximum(o, 0.0)
    o_ref[...] = o


def _mm(agg, x, wl, b, wr, relu):
    n, din = x.shape
    dout = wl.shape[1]
    rblk = 2000
    return pl.pallas_call(
        functools.partial(_mm_body, relu=relu),
        grid=(n // rblk,),
        in_specs=[
            pl.BlockSpec((rblk, din), lambda i: (i, 0)),
            pl.BlockSpec((rblk, din), lambda i: (i, 0)),
            pl.BlockSpec((din, dout), lambda i: (0, 0)),
            pl.BlockSpec((dout,), lambda i: (0,)),
            pl.BlockSpec((din, dout), lambda i: (0, 0)),
        ],
        out_specs=pl.BlockSpec((rblk, dout), lambda i: (i, 0)),
        out_shape=jax.ShapeDtypeStruct((n, dout), jnp.float32),
    )(agg, x, wl, b, wr)


def kernel(x, edge_index, W1_l, b1_l, W1_r, W2_l, b2_l, W2_r):
    pad = EPAD - E
    src = jnp.concatenate([edge_index[0], jnp.broadcast_to(edge_index[0, -1:], (pad,))])
    dst = jnp.concatenate([edge_index[1], jnp.broadcast_to(edge_index[1, -1:], (pad,))])
    agg1, lists, tots = _seg_max_scan(x, src, dst)
    h = _mm(agg1[:N], x, W1_l, b1_l, W1_r, relu=True)
    agg2 = _seg_max_replay(h, lists, tots)[:N]
    out = _mm(agg2, h, W2_l, b2_l, W2_r, relu=False)
    return out
